# baseline (speedup 1.0000x reference)
import functools

import jax
import jax.numpy as jnp
from jax import lax
from jax.experimental import pallas as pl
from jax.experimental.pallas import tpu as pltpu
from jax.experimental.pallas import tpu_sc as plsc

BATCH = 4096
FIELDS = 26
EMBED_DIM = 64
NB = BATCH * FIELDS
NC = 2
NS = 16
NW = NC * NS
BPW = NB // NW
CHUNK = 128
NCHUNK = BPW // CHUNK
NBUF = 4

_mesh = plsc.VectorSubcoreMesh(core_axis_name="c", subcore_axis_name="s")


@functools.partial(
    pl.kernel,
    mesh=_mesh,
    out_type=jax.ShapeDtypeStruct((NB, EMBED_DIM), jnp.float32),
    scratch_types=[
        pltpu.VMEM((NCHUNK, CHUNK), jnp.int32),
        pltpu.VMEM((NBUF, CHUNK, EMBED_DIM), jnp.float32),
        pltpu.SemaphoreType.DMA,
        pltpu.SemaphoreType.DMA,
    ],
    compiler_params=pltpu.CompilerParams(use_tc_tiling_on_sc=False),
)
def _embed_gather(idx_hbm, table_hbm, out_hbm, idx_v, rows_v, gsem, ssem):
    wid = lax.axis_index("s") * NC + lax.axis_index("c")
    base = wid * BPW
    pltpu.sync_copy(idx_hbm.at[wid], idx_v)

    gathers = [None] * NCHUNK
    scatters = [None] * NCHUNK
    for g in range(min(NBUF - 1, NCHUNK)):
        gathers[g] = pltpu.async_copy(
            table_hbm.at[idx_v.at[g]], rows_v.at[g % NBUF], gsem)
    for g in range(NCHUNK):
        gathers[g].wait()
        scatters[g] = pltpu.async_copy(
            rows_v.at[g % NBUF],
            out_hbm.at[pl.ds(base + g * CHUNK, CHUNK)],
            ssem)
        nxt = g + NBUF - 1
        if nxt < NCHUNK:
            prev = nxt - NBUF
            if prev >= 0:
                scatters[prev].wait()
            gathers[nxt] = pltpu.async_copy(
                table_hbm.at[idx_v.at[nxt]], rows_v.at[nxt % NBUF], gsem)
    for g in range(max(0, NCHUNK - NBUF), NCHUNK):
        scatters[g].wait()


def kernel(feat_ids, table):
    idx = feat_ids.astype(jnp.int32).reshape(NW, NCHUNK, CHUNK)
    out = _embed_gather(idx, table)
    return out.reshape(BATCH, FIELDS, EMBED_DIM)
